# Initial kernel scaffold; baseline (speedup 1.0000x reference)
import jax
import jax.numpy as jnp
from jax.experimental import pallas as pl


def kernel(boxes, scores):
    raise NotImplementedError("write your pallas kernel here")


# speedup vs baseline: 22.7169x; 22.7169x over previous
import jax
import jax.numpy as jnp
from jax.experimental import pallas as pl

N = 5000
ROWS, COLS = 8, 640
PADN = ROWS * COLS
SCORE_TH = 0.05
NMS_TH = 0.5
MAX_DET = 100
NEG_INF = float("-inf")


def _nms_kernel(s_ref, x1_ref, y1_ref, x2_ref, y2_ref, out_ref):
    x1 = x1_ref[...]
    y1 = y1_ref[...]
    x2 = x2_ref[...]
    y2 = y2_ref[...]
    s_in = s_ref[...]
    s0 = jnp.where(s_in > SCORE_TH, s_in, NEG_INF)
    area = (x2 - x1) * (y2 - y1)
    ridx = jax.lax.broadcasted_iota(jnp.int32, (ROWS, COLS), 0)
    cidx = jax.lax.broadcasted_iota(jnp.int32, (ROWS, COLS), 1)
    gidx = ridx * COLS + cidx
    lane = jax.lax.broadcasted_iota(jnp.int32, (1, 128), 1)

    def body(i, s):
        maxval = jnp.max(s)
        cand = jnp.where(s >= maxval, gidx, PADN)
        idx = jnp.min(cand)
        onehot = gidx == idx
        bx1 = jnp.sum(jnp.where(onehot, x1, 0.0))
        by1 = jnp.sum(jnp.where(onehot, y1, 0.0))
        bx2 = jnp.sum(jnp.where(onehot, x2, 0.0))
        by2 = jnp.sum(jnp.where(onehot, y2, 0.0))
        a1 = (bx2 - bx1) * (by2 - by1)
        ix1 = jnp.maximum(bx1, x1)
        iy1 = jnp.maximum(by1, y1)
        ix2 = jnp.minimum(bx2, x2)
        iy2 = jnp.minimum(by2, y2)
        inter = jnp.maximum(ix2 - ix1, 0.0) * jnp.maximum(iy2 - iy1, 0.0)
        iou = inter / (a1 + area - inter + 1e-9)
        sup = (iou > NMS_TH) | onehot
        s_new = jnp.where(sup, NEG_INF, s)
        valid = maxval > NEG_INF
        row = jnp.where(lane == 0, bx1,
              jnp.where(lane == 1, by1,
              jnp.where(lane == 2, bx2,
              jnp.where(lane == 3, by2,
              jnp.where(lane == 4, maxval, 0.0)))))
        row = jnp.where(valid, row, 0.0)
        out_ref[pl.ds(i, 1), :] = row
        return s_new

    jax.lax.fori_loop(0, MAX_DET, body, s0)


def kernel(boxes, scores):
    pad = PADN - N
    s2d = jnp.pad(scores, (0, pad)).reshape(ROWS, COLS)
    bpad = jnp.pad(boxes, ((0, pad), (0, 0)))
    x1 = bpad[:, 0].reshape(ROWS, COLS)
    y1 = bpad[:, 1].reshape(ROWS, COLS)
    x2 = bpad[:, 2].reshape(ROWS, COLS)
    y2 = bpad[:, 3].reshape(ROWS, COLS)
    out = pl.pallas_call(
        _nms_kernel,
        out_shape=jax.ShapeDtypeStruct((MAX_DET, 128), jnp.float32),
    )(s2d, x1, y1, x2, y2)
    return out[:, :5]
